# baseline (speedup 1.0000x reference)
import functools

import jax
import jax.numpy as jnp
from jax import lax
from jax.experimental import pallas as pl
from jax.experimental.pallas import tpu as pltpu
from jax.experimental.pallas import tpu_sc as plsc

HIDDEN = 2048
NUM_TOKENS = 4096
NUM_ROWS = 16384

NUM_CORES = 2
NUM_SUBCORES = 16
NUM_WORKERS = NUM_CORES * NUM_SUBCORES
PER_WORKER = NUM_TOKENS // NUM_WORKERS
CHUNK = 32
NUM_CHUNKS = PER_WORKER // CHUNK


def _scatter_body(image_hbm, idx_hbm, src_hbm, out_hbm, idx_v, src_v, rows_v, sem):
    wid = lax.axis_index("s") * NUM_CORES + lax.axis_index("c")
    pltpu.sync_copy(idx_hbm.at[wid], idx_v)
    pltpu.sync_copy(src_hbm.at[wid], src_v)
    for c in range(NUM_CHUNKS):
        pltpu.async_copy(image_hbm.at[src_v.at[c]], rows_v, sem).wait()
        pltpu.async_copy(rows_v, out_hbm.at[idx_v.at[c]], sem).wait()


@functools.cache
def _get_scatter():
    return pl.kernel(
        _scatter_body,
        out_type=(),
        mesh=plsc.VectorSubcoreMesh(
            core_axis_name="c",
            subcore_axis_name="s",
            num_cores=NUM_CORES,
            num_subcores=NUM_SUBCORES,
        ),
        scratch_types=[
            pltpu.VMEM((NUM_CHUNKS, CHUNK), jnp.int32),
            pltpu.VMEM((NUM_CHUNKS, CHUNK), jnp.int32),
            pltpu.VMEM((CHUNK, HIDDEN), jnp.float32),
            pltpu.SemaphoreType.DMA,
        ],
    )


def kernel(image_embeddings, text_embeddings, vision_indices):
    batch, seq_len, hidden = text_embeddings.shape
    flat = jnp.reshape(text_embeddings, (batch * seq_len, hidden))
    idx = vision_indices.astype(jnp.int32)
    src = (jnp.searchsorted(idx, idx, side="right") - 1).astype(jnp.int32)
    idx3 = jnp.reshape(idx, (NUM_WORKERS, NUM_CHUNKS, CHUNK))
    src3 = jnp.reshape(src, (NUM_WORKERS, NUM_CHUNKS, CHUNK))
    out_ref = jax.new_ref(flat)
    _get_scatter()(image_embeddings, idx3, src3, out_ref)
    return jnp.reshape(out_ref[...], (batch, seq_len, hidden))


# speedup vs baseline: 2.0828x; 2.0828x over previous
import functools

import jax
import jax.numpy as jnp
from jax import lax
from jax.experimental import pallas as pl
from jax.experimental.pallas import tpu as pltpu
from jax.experimental.pallas import tpu_sc as plsc

HIDDEN = 2048
NUM_TOKENS = 4096
NUM_ROWS = 16384

NUM_CORES = 2
NUM_SUBCORES = 16
NUM_WORKERS = NUM_CORES * NUM_SUBCORES
PER_WORKER = NUM_TOKENS // NUM_WORKERS
CHUNK = 16
NUM_CHUNKS = PER_WORKER // CHUNK


def _scatter_body(image_hbm, idx_hbm, src_hbm, out_hbm, idx_v, src_v,
                  rows_a, rows_b, gsem, ssem):
    wid = lax.axis_index("s") * NUM_CORES + lax.axis_index("c")
    pltpu.sync_copy(idx_hbm.at[wid], idx_v)
    pltpu.sync_copy(src_hbm.at[wid], src_v)
    bufs = (rows_a, rows_b)
    gathers = [None] * NUM_CHUNKS
    scatters = [None] * NUM_CHUNKS
    gathers[0] = pltpu.async_copy(image_hbm.at[src_v.at[0]], bufs[0], gsem)
    for c in range(NUM_CHUNKS):
        if c + 1 < NUM_CHUNKS:
            if c >= 1:
                scatters[c - 1].wait()
            gathers[c + 1] = pltpu.async_copy(
                image_hbm.at[src_v.at[c + 1]], bufs[(c + 1) % 2], gsem)
        gathers[c].wait()
        scatters[c] = pltpu.async_copy(bufs[c % 2], out_hbm.at[idx_v.at[c]], ssem)
    scatters[NUM_CHUNKS - 1].wait()


@functools.cache
def _get_scatter():
    return pl.kernel(
        _scatter_body,
        out_type=(),
        mesh=plsc.VectorSubcoreMesh(
            core_axis_name="c",
            subcore_axis_name="s",
            num_cores=NUM_CORES,
            num_subcores=NUM_SUBCORES,
        ),
        scratch_types=[
            pltpu.VMEM((NUM_CHUNKS, CHUNK), jnp.int32),
            pltpu.VMEM((NUM_CHUNKS, CHUNK), jnp.int32),
            pltpu.VMEM((CHUNK, HIDDEN), jnp.float32),
            pltpu.VMEM((CHUNK, HIDDEN), jnp.float32),
            pltpu.SemaphoreType.DMA,
            pltpu.SemaphoreType.DMA,
        ],
    )


def kernel(image_embeddings, text_embeddings, vision_indices):
    batch, seq_len, hidden = text_embeddings.shape
    flat = jnp.reshape(text_embeddings, (batch * seq_len, hidden))
    idx = vision_indices.astype(jnp.int32)
    iota = jnp.arange(NUM_TOKENS, dtype=jnp.int32)
    nxt = jnp.concatenate([idx[1:], jnp.full((1,), -1, jnp.int32)])
    src = lax.cummin(
        jnp.where(idx != nxt, iota, jnp.int32(NUM_TOKENS)), axis=0, reverse=True
    ).astype(jnp.int32)
    idx3 = jnp.reshape(idx, (NUM_WORKERS, NUM_CHUNKS, CHUNK))
    src3 = jnp.reshape(src, (NUM_WORKERS, NUM_CHUNKS, CHUNK))
    out_ref = jax.new_ref(flat)
    _get_scatter()(image_embeddings, idx3, src3, out_ref)
    return jnp.reshape(out_ref[...], (batch, seq_len, hidden))
